# baseline (speedup 1.0000x reference)
import functools

import jax
import jax.numpy as jnp
from jax import lax
from jax.experimental import pallas as pl
from jax.experimental.pallas import tpu as pltpu
from jax.experimental.pallas import tpu_sc as plsc

N = 2
CH = 64
H = 64
W = 64
NH = 4
CHUNK = 144
C = 16
HB = 56
L = 2 * H * W
PAD = 16
K = (L + PAD) // CHUNK
LP = K * CHUNK
BH = N * NH
PB = 256
NPB = L // PB
HBP = 64
EB = 512
NIMG = 2 * N
KK = 9 * CH
OC = 128

FC0 = 0
Y0 = 144
X0 = 208
XN0 = 224


def _embed_body(p_ref, w3_ref, w1_ref, b1_ref, w2_ref, b2_ref, rot_ref,
                tok_ref, oh_ref):
    pb = p_ref[0]
    full = jax.lax.dot_general(
        pb, w3_ref[...], (((0,), (1,)), ((), ())),
        preferred_element_type=jnp.float32)
    xe = full[:, :C]
    ye = full[:, C:C + CH]
    fe = full[:, C + CH:]
    hdn = jax.nn.relu(jax.lax.dot_general(
        fe, w1_ref[...], (((1,), (1,)), ((), ())),
        preferred_element_type=jnp.float32) + b1_ref[...])
    fco = jax.lax.dot_general(
        hdn, w2_ref[...], (((1,), (1,)), ((), ())),
        preferred_element_type=jnp.float32) + b2_ref[...]
    nrm = jnp.maximum(
        jnp.sqrt(jnp.sum(xe * xe, axis=1, keepdims=True)), 5e-05)
    tok_ref[0] = jnp.concatenate(
        [fco, ye, xe, xe / nrm, jnp.zeros((EB, PAYW - XN0 - C), jnp.float32)],
        axis=1)
    rot = jax.lax.dot_general(
        xe, rot_ref[...], (((1,), (0,)), ((), ())),
        preferred_element_type=jnp.float32)
    lanes = jax.lax.broadcasted_iota(jnp.int32, (EB, NH * HBP), 1)
    rot = rot + jnp.where(lanes % HBP >= HB, -1e30, 0.0)
    mlt = (jax.lax.broadcasted_iota(jnp.int32, (HBP, HBP), 0) <
           jax.lax.broadcasted_iota(jnp.int32, (HBP, HBP), 1)
           ).astype(jnp.bfloat16)
    for h in range(NH):
        rb = rot[:, h * HBP:(h + 1) * HBP]
        m = jnp.max(rb, axis=1, keepdims=True)
        eqb = rb == m
        cnt = jax.lax.dot_general(
            eqb.astype(jnp.bfloat16), mlt, (((1,), (0,)), ((), ())),
            preferred_element_type=jnp.float32)
        oh_ref[h, 0] = (eqb & (cnt == 0.0)).astype(jnp.float32)


def _embed(patches, w3, fc_w1, fc_b1, fc_w2, fc_b2, rotp):
    zero2 = lambda i, t: (0, 0)
    return pl.pallas_call(
        _embed_body,
        grid=(NIMG, (H * W) // EB),
        in_specs=[
            pl.BlockSpec((1, KK, EB), lambda i, t: (i, 0, t)),
            pl.BlockSpec((CHUNK, KK), zero2),
            pl.BlockSpec((CHUNK, CH), zero2),
            pl.BlockSpec((1, CHUNK), zero2),
            pl.BlockSpec((CHUNK, CHUNK), zero2),
            pl.BlockSpec((1, CHUNK), zero2),
            pl.BlockSpec((C, NH * HBP), zero2),
        ],
        out_specs=[
            pl.BlockSpec((1, EB, PAYW), lambda i, t: (i, t, 0)),
            pl.BlockSpec((NH, 1, EB, HBP), lambda i, t: (0, i, t, 0)),
        ],
        out_shape=[
            jax.ShapeDtypeStruct((NIMG, H * W, PAYW), jnp.float32),
            jax.ShapeDtypeStruct((NH, NIMG, H * W, HBP), jnp.float32),
        ],
    )(patches, w3, fc_w1, fc_b1, fc_w2, fc_b2, rotp)


def _pos_body(oh_ref, pos_ref):
    b = pl.program_id(0)
    h = pl.program_id(1)
    ri = jax.lax.broadcasted_iota(jnp.int32, (PB, PB), 0)
    ci = jax.lax.broadcasted_iota(jnp.int32, (PB, PB), 1)
    t_le_bf = (ri >= ci).astype(jnp.bfloat16)
    vi = jax.lax.broadcasted_iota(jnp.int32, (HBP, HBP), 0)
    vj = jax.lax.broadcasted_iota(jnp.int32, (HBP, HBP), 1)
    t_lt = (vi < vj).astype(jnp.float32)

    oh_all = oh_ref[0, 0]
    tot = jnp.sum(oh_all, axis=0, keepdims=True)
    start = jax.lax.dot_general(
        tot, t_lt, (((1,), (0,)), ((), ())),
        precision=jax.lax.Precision.HIGHEST,
        preferred_element_type=jnp.float32)
    base = ((b * NH + h) * LP).astype(jnp.int32)

    def blk(t, run):
        ob = oh_ref[0, 0, pl.ds(t * PB, PB), :]
        rinc = jax.lax.dot_general(
            t_le_bf, ob.astype(jnp.bfloat16), (((1,), (0,)), ((), ())),
            preferred_element_type=jnp.float32)
        bias = run + start
        posf = jnp.sum(ob * (rinc - ob + bias), axis=1, keepdims=True)
        pos_ref[0, 0, pl.ds(t * PB, PB), :] = posf.astype(jnp.int32) + base
        return run + rinc[PB - 1:PB, :]

    jax.lax.fori_loop(0, NPB, blk, jnp.zeros((1, HBP), jnp.float32))


def _positions(onehot):
    return pl.pallas_call(
        _pos_body,
        grid=(N, NH),
        in_specs=[pl.BlockSpec((1, 1, L, HBP), lambda b, h: (h, b, 0, 0))],
        out_specs=pl.BlockSpec((1, 1, L, 1), lambda b, h: (b, h, 0, 0)),
        out_shape=jax.ShapeDtypeStruct((N, NH, L, 1), jnp.int32),
    )(onehot)


def _attn_body(srt_ref, out_ref):
    ident = (jax.lax.broadcasted_iota(jnp.int32, (CHUNK, CHUNK), 0) ==
             jax.lax.broadcasted_iota(jnp.int32, (CHUNK, CHUNK), 1)
             ).astype(jnp.float32)

    def compute(a3, q, k):
        xm = a3[:, XN0:XN0 + C]
        yk = a3[:, Y0:Y0 + CH]
        fc3 = a3[:, FC0:FC0 + CHUNK]
        raw_t = jax.lax.dot_general(
            xm, q, (((1,), (1,)), ((), ())),
            preferred_element_type=jnp.float32) + fc3
        m = jnp.max(raw_t, axis=0, keepdims=True)
        e = jnp.exp(raw_t - m)
        s = jnp.sum(e, axis=0, keepdims=True)
        en = e * (1.0 / s)
        ret = jax.lax.dot_general(
            en, yk, (((0,), (0,)), ((), ())),
            preferred_element_type=jnp.float32)
        bs_row = m + jnp.log(s)
        bs_col = jax.lax.dot_general(
            ident, bs_row, (((1,), (1,)), ((), ())),
            preferred_element_type=jnp.float32)
        payload = jnp.concatenate(
            [ret, jnp.broadcast_to(bs_col, (CHUNK, OC - CH))], axis=1)
        out_ref[0, pl.ds(k * CHUNK, CHUNK), :] = payload

    c56 = jnp.concatenate(
        [srt_ref[0, pl.ds((K - 1) * CHUNK, CHUNK - PAD), :],
         srt_ref[0, pl.ds(L - PAD, PAD), :]], axis=0)
    c0 = srt_ref[0, pl.ds(0, CHUNK), :]
    c1 = srt_ref[0, pl.ds(CHUNK, CHUNK), :]
    c54 = srt_ref[0, pl.ds((K - 3) * CHUNK, CHUNK), :]
    c55 = srt_ref[0, pl.ds((K - 2) * CHUNK, CHUNK), :]
    compute(jnp.concatenate([c56, c0, c1], axis=0), c0[:, X0:X0 + C], 0)
    compute(jnp.concatenate([c54, c55, c56], axis=0), c55[:, X0:X0 + C], K - 2)
    compute(jnp.concatenate([c55, c56, c0], axis=0), c56[:, X0:X0 + C], K - 1)

    def chunk_pair(i, _):
        k = 1 + 2 * i
        a3 = srt_ref[0, pl.ds((k - 1) * CHUNK, 3 * CHUNK), :]
        compute(a3, a3[CHUNK:2 * CHUNK, X0:X0 + C], k)
        b3 = srt_ref[0, pl.ds(k * CHUNK, 3 * CHUNK), :]
        compute(b3, b3[CHUNK:2 * CHUNK, X0:X0 + C], k + 1)
        return 0

    jax.lax.fori_loop(0, (K - 3) // 2, chunk_pair, 0)


def _chunked_attention(srt):
    return pl.pallas_call(
        _attn_body,
        grid=(BH,),
        in_specs=[pl.BlockSpec((1, LP, PAYW), lambda b: (b, 0, 0))],
        out_specs=pl.BlockSpec((1, LP, OC), lambda b: (b, 0, 0)),
        out_shape=jax.ShapeDtypeStruct((BH, LP, OC), jnp.float32),
    )(srt)


def _conv(x, w):
    return jax.lax.conv_general_dilated(
        x, w, (1, 1), 'SAME', dimension_numbers=('NCHW', 'OIHW', 'NCHW'))


NW = 32
SLICE = L // NW
IDXR = SLICE // 128
PAYW = 256

_SC_MESH = plsc.VectorSubcoreMesh(core_axis_name="c", subcore_axis_name="s")


def _sc_scatter_body(tok, pos, srt, idx_v, row_v, sem):
    wid = lax.axis_index("s") * 2 + lax.axis_index("c")
    for b in range(N):
        pltpu.sync_copy(tok.at[pl.ds(b * L + wid * SLICE, SLICE)], row_v)
        for h in range(NH):
            pltpu.sync_copy(pos.at[b, h, pl.ds(wid * IDXR, IDXR)], idx_v)
            handles = [
                pltpu.async_copy(row_v.at[pl.ds(j * 128, 128)],
                                 srt.at[idx_v.at[j]], sem)
                for j in range(IDXR)
            ]
            for hd in handles:
                hd.wait()


def _sc_scatter(tok, pos4):
    return pl.kernel(
        _sc_scatter_body,
        out_type=jax.ShapeDtypeStruct((BH * LP, PAYW), jnp.float32),
        mesh=_SC_MESH,
        scratch_types=[
            pltpu.VMEM((IDXR, 128), jnp.int32),
            pltpu.VMEM((SLICE, PAYW), jnp.float32),
            pltpu.SemaphoreType.DMA,
        ],
    )(tok, pos4)


def _sc_gather_body(att, pos, retf, idx_v, rows_v, sem):
    wid = lax.axis_index("s") * 2 + lax.axis_index("c")
    for b in range(N):
        for h in range(NH):
            pltpu.sync_copy(pos.at[b, h, pl.ds(wid * IDXR, IDXR)], idx_v)
            for j in range(IDXR):
                pltpu.async_copy(att.at[idx_v.at[j]], rows_v, sem).wait()
                dst = (b * NH + h) * L + wid * SLICE + j * 128
                pltpu.sync_copy(rows_v, retf.at[pl.ds(dst, 128)])


def _sc_gather(att, pos4):
    return pl.kernel(
        _sc_gather_body,
        out_type=jax.ShapeDtypeStruct((N * NH * L, OC), jnp.float32),
        mesh=_SC_MESH,
        scratch_types=[
            pltpu.VMEM((IDXR, 128), jnp.int32),
            pltpu.VMEM((128, OC), jnp.float32),
            pltpu.SemaphoreType.DMA,
        ],
    )(att, pos4)


CB = 512


def _comb_body(retf_ref, out_ref):
    lane = lax.broadcasted_iota(jnp.int32, (CB, OC), 1)
    msk = (lane == CH).astype(jnp.float32)
    rows = [retf_ref[0, h] for h in range(NH)]
    bs = [jnp.sum(r * msk, axis=1, keepdims=True) for r in rows]
    m = bs[0]
    for h in range(1, NH):
        m = jnp.maximum(m, bs[h])
    es = [jnp.exp(b - m) for b in bs]
    s = es[0]
    for h in range(1, NH):
        s = s + es[h]
    acc = jnp.zeros((CB, CH), jnp.float32)
    for h in range(NH):
        acc = acc + rows[h][:, :CH] * (es[h] / s)
    out_ref[0] = acc


def _combine(retf):
    return pl.pallas_call(
        _comb_body,
        grid=(N, L // CB),
        in_specs=[pl.BlockSpec((1, NH, CB, OC), lambda b, t: (b, 0, t, 0))],
        out_specs=pl.BlockSpec((1, CB, CH), lambda b, t: (b, t, 0)),
        out_shape=jax.ShapeDtypeStruct((N, L, CH), jnp.float32),
    )(retf)


def kernel(input1, input2, w_match, w_asm, w_asm_fc, fc_w1, fc_b1, fc_w2,
           fc_b2, rotations):
    n = input1.shape[0]
    hw = H * W

    imgs = jnp.stack([input1, input2], axis=1).reshape(NIMG, CH, H, W)
    ip = jnp.pad(imgs, ((0, 0), (0, 0), (1, 1), (1, 1)))
    patches = jnp.stack(
        [ip[:, :, ky:ky + H, kx:kx + W]
         for ky in range(3) for kx in range(3)],
        axis=2).reshape(NIMG, KK, hw)
    w3 = jnp.concatenate(
        [w_match.reshape(C, KK), w_asm.reshape(CH, KK),
         w_asm_fc.reshape(CH, KK)], axis=0)
    rotp = jnp.pad(rotations, ((0, 0), (0, 0), (0, HBP - HB))).reshape(
        C, NH * HBP)

    tok, onehot = _embed(patches, w3, fc_w1, fc_b1.reshape(1, CHUNK), fc_w2,
                         fc_b2.reshape(1, CHUNK), rotp)

    pos4 = _positions(onehot.reshape(NH, N, L, HBP)).reshape(
        N, NH, L // 128, 128)

    srt = _sc_scatter(tok.reshape(n * L, PAYW), pos4)

    att = _chunked_attention(srt.reshape(BH, LP, PAYW))

    retf = _sc_gather(att.reshape(BH * LP, OC), pos4)
    ret = _combine(retf.reshape(n, NH, L, OC))

    out1 = ret[:, :hw, :].transpose(0, 2, 1).reshape(n, CH, H, W) + input1
    out2 = ret[:, hw:, :].transpose(0, 2, 1).reshape(n, CH, H, W) + input2
    return (out1, out2)


# speedup vs baseline: 1.0724x; 1.0724x over previous
import functools

import jax
import jax.numpy as jnp
from jax import lax
from jax.experimental import pallas as pl
from jax.experimental.pallas import tpu as pltpu
from jax.experimental.pallas import tpu_sc as plsc

N = 2
CH = 64
H = 64
W = 64
NH = 4
CHUNK = 144
C = 16
HB = 56
L = 2 * H * W
PAD = 16
K = (L + PAD) // CHUNK
LP = K * CHUNK
BH = N * NH
PB = 256
NPB = L // PB
HBP = 64
EB = 512
NIMG = 2 * N
KK = 9 * CH
OC = 128

FC0 = 0
Y0 = 144
X0 = 208
XN0 = 224


def _embed_body(p_ref, w3_ref, w1_ref, b1_ref, w2_ref, b2_ref, rot_ref,
                tok_ref, oh_ref):
    pb = p_ref[0]
    full = jax.lax.dot_general(
        pb, w3_ref[...], (((0,), (1,)), ((), ())),
        preferred_element_type=jnp.float32)
    xe = full[:, :C]
    ye = full[:, C:C + CH]
    fe = full[:, C + CH:]
    hdn = jax.nn.relu(jax.lax.dot_general(
        fe, w1_ref[...], (((1,), (1,)), ((), ())),
        preferred_element_type=jnp.float32) + b1_ref[...])
    fco = jax.lax.dot_general(
        hdn, w2_ref[...], (((1,), (1,)), ((), ())),
        preferred_element_type=jnp.float32) + b2_ref[...]
    nrm = jnp.maximum(
        jnp.sqrt(jnp.sum(xe * xe, axis=1, keepdims=True)), 5e-05)
    tok_ref[0] = jnp.concatenate(
        [fco, ye, xe, xe / nrm, jnp.zeros((EB, PAYW - XN0 - C), jnp.float32)],
        axis=1)
    rot = jax.lax.dot_general(
        xe, rot_ref[...], (((1,), (0,)), ((), ())),
        preferred_element_type=jnp.float32)
    lanes = jax.lax.broadcasted_iota(jnp.int32, (EB, NH * HBP), 1)
    rot = rot + jnp.where(lanes % HBP >= HB, -1e30, 0.0)
    mlt = (jax.lax.broadcasted_iota(jnp.int32, (HBP, HBP), 0) <
           jax.lax.broadcasted_iota(jnp.int32, (HBP, HBP), 1)
           ).astype(jnp.bfloat16)
    for h in range(NH):
        rb = rot[:, h * HBP:(h + 1) * HBP]
        m = jnp.max(rb, axis=1, keepdims=True)
        eqb = rb == m
        cnt = jax.lax.dot_general(
            eqb.astype(jnp.bfloat16), mlt, (((1,), (0,)), ((), ())),
            preferred_element_type=jnp.float32)
        oh_ref[h, 0] = (eqb & (cnt == 0.0)).astype(jnp.float32)


def _embed(patches, w3, fc_w1, fc_b1, fc_w2, fc_b2, rotp):
    zero2 = lambda i, t: (0, 0)
    return pl.pallas_call(
        _embed_body,
        grid=(NIMG, (H * W) // EB),
        in_specs=[
            pl.BlockSpec((1, KK, EB), lambda i, t: (i, 0, t)),
            pl.BlockSpec((CHUNK, KK), zero2),
            pl.BlockSpec((CHUNK, CH), zero2),
            pl.BlockSpec((1, CHUNK), zero2),
            pl.BlockSpec((CHUNK, CHUNK), zero2),
            pl.BlockSpec((1, CHUNK), zero2),
            pl.BlockSpec((C, NH * HBP), zero2),
        ],
        out_specs=[
            pl.BlockSpec((1, EB, PAYW), lambda i, t: (i, t, 0)),
            pl.BlockSpec((NH, 1, EB, HBP), lambda i, t: (0, i, t, 0)),
        ],
        out_shape=[
            jax.ShapeDtypeStruct((NIMG, H * W, PAYW), jnp.float32),
            jax.ShapeDtypeStruct((NH, NIMG, H * W, HBP), jnp.float32),
        ],
    )(patches, w3, fc_w1, fc_b1, fc_w2, fc_b2, rotp)


def _pos_body(oh_ref, pos_ref):
    b = pl.program_id(0)
    h = pl.program_id(1)
    ri = jax.lax.broadcasted_iota(jnp.int32, (PB, PB), 0)
    ci = jax.lax.broadcasted_iota(jnp.int32, (PB, PB), 1)
    t_le_bf = (ri >= ci).astype(jnp.bfloat16)
    vi = jax.lax.broadcasted_iota(jnp.int32, (HBP, HBP), 0)
    vj = jax.lax.broadcasted_iota(jnp.int32, (HBP, HBP), 1)
    t_lt = (vi < vj).astype(jnp.float32)

    oh_all = oh_ref[0, 0]
    tot = jnp.sum(oh_all, axis=0, keepdims=True)
    start = jax.lax.dot_general(
        tot, t_lt, (((1,), (0,)), ((), ())),
        precision=jax.lax.Precision.HIGHEST,
        preferred_element_type=jnp.float32)
    base = ((b * NH + h) * LP).astype(jnp.int32)

    def blk(t, run):
        ob = oh_ref[0, 0, pl.ds(t * PB, PB), :]
        rinc = jax.lax.dot_general(
            t_le_bf, ob.astype(jnp.bfloat16), (((1,), (0,)), ((), ())),
            preferred_element_type=jnp.float32)
        bias = run + start
        posf = jnp.sum(ob * (rinc - ob + bias), axis=1, keepdims=True)
        pos_ref[0, 0, pl.ds(t * PB, PB), :] = posf.astype(jnp.int32) + base
        return run + rinc[PB - 1:PB, :]

    jax.lax.fori_loop(0, NPB, blk, jnp.zeros((1, HBP), jnp.float32))


def _positions(onehot):
    return pl.pallas_call(
        _pos_body,
        grid=(N, NH),
        in_specs=[pl.BlockSpec((1, 1, L, HBP), lambda b, h: (b, h, 0, 0))],
        out_specs=pl.BlockSpec((1, 1, L, 1), lambda b, h: (b, h, 0, 0)),
        out_shape=jax.ShapeDtypeStruct((N, NH, L, 1), jnp.int32),
    )(onehot)


def _attn_body(srt_ref, out_ref):
    ident = (jax.lax.broadcasted_iota(jnp.int32, (CHUNK, CHUNK), 0) ==
             jax.lax.broadcasted_iota(jnp.int32, (CHUNK, CHUNK), 1)
             ).astype(jnp.float32)

    def compute(a3, q, k):
        xm = a3[:, XN0:XN0 + C]
        yk = a3[:, Y0:Y0 + CH]
        fc3 = a3[:, FC0:FC0 + CHUNK]
        raw_t = jax.lax.dot_general(
            xm, q, (((1,), (1,)), ((), ())),
            preferred_element_type=jnp.float32) + fc3
        m = jnp.max(raw_t, axis=0, keepdims=True)
        e = jnp.exp(raw_t - m)
        s = jnp.sum(e, axis=0, keepdims=True)
        en = e * (1.0 / s)
        ret = jax.lax.dot_general(
            en, yk, (((0,), (0,)), ((), ())),
            preferred_element_type=jnp.float32)
        bs_row = m + jnp.log(s)
        bs_col = jax.lax.dot_general(
            ident, bs_row, (((1,), (1,)), ((), ())),
            preferred_element_type=jnp.float32)
        payload = jnp.concatenate(
            [ret, jnp.broadcast_to(bs_col, (CHUNK, OC - CH))], axis=1)
        out_ref[0, pl.ds(k * CHUNK, CHUNK), :] = payload

    c56 = jnp.concatenate(
        [srt_ref[0, pl.ds((K - 1) * CHUNK, CHUNK - PAD), :],
         srt_ref[0, pl.ds(L - PAD, PAD), :]], axis=0)
    c0 = srt_ref[0, pl.ds(0, CHUNK), :]
    c1 = srt_ref[0, pl.ds(CHUNK, CHUNK), :]
    c54 = srt_ref[0, pl.ds((K - 3) * CHUNK, CHUNK), :]
    c55 = srt_ref[0, pl.ds((K - 2) * CHUNK, CHUNK), :]
    compute(jnp.concatenate([c56, c0, c1], axis=0), c0[:, X0:X0 + C], 0)
    compute(jnp.concatenate([c54, c55, c56], axis=0), c55[:, X0:X0 + C], K - 2)
    compute(jnp.concatenate([c55, c56, c0], axis=0), c56[:, X0:X0 + C], K - 1)

    def chunk_pair(i, _):
        k = 1 + 2 * i
        a3 = srt_ref[0, pl.ds((k - 1) * CHUNK, 3 * CHUNK), :]
        compute(a3, a3[CHUNK:2 * CHUNK, X0:X0 + C], k)
        b3 = srt_ref[0, pl.ds(k * CHUNK, 3 * CHUNK), :]
        compute(b3, b3[CHUNK:2 * CHUNK, X0:X0 + C], k + 1)
        return 0

    jax.lax.fori_loop(0, (K - 3) // 2, chunk_pair, 0)


def _chunked_attention(srt):
    return pl.pallas_call(
        _attn_body,
        grid=(BH,),
        in_specs=[pl.BlockSpec((1, LP, PAYW), lambda b: (b, 0, 0))],
        out_specs=pl.BlockSpec((1, LP, OC), lambda b: (b, 0, 0)),
        out_shape=jax.ShapeDtypeStruct((BH, LP, OC), jnp.float32),
    )(srt)


def _conv(x, w):
    return jax.lax.conv_general_dilated(
        x, w, (1, 1), 'SAME', dimension_numbers=('NCHW', 'OIHW', 'NCHW'))


NW = 32
SLICE = L // NW
IDXR = SLICE // 128
PAYW = 256

_SC_MESH = plsc.VectorSubcoreMesh(core_axis_name="c", subcore_axis_name="s")


def _sc_scatter_body(tok, pos, srt, idx_v, row_v, sem):
    wid = lax.axis_index("s") * 2 + lax.axis_index("c")
    for b in range(N):
        pltpu.sync_copy(tok.at[pl.ds(b * L + wid * SLICE, SLICE)], row_v)
        for h in range(NH):
            pltpu.sync_copy(pos.at[b, h, pl.ds(wid * IDXR, IDXR)], idx_v)
            handles = [
                pltpu.async_copy(row_v.at[pl.ds(j * 128, 128)],
                                 srt.at[idx_v.at[j]], sem)
                for j in range(IDXR)
            ]
            for hd in handles:
                hd.wait()


def _sc_scatter(tok, pos4):
    return pl.kernel(
        _sc_scatter_body,
        out_type=jax.ShapeDtypeStruct((BH * LP, PAYW), jnp.float32),
        mesh=_SC_MESH,
        scratch_types=[
            pltpu.VMEM((IDXR, 128), jnp.int32),
            pltpu.VMEM((SLICE, PAYW), jnp.float32),
            pltpu.SemaphoreType.DMA,
        ],
    )(tok, pos4)


def _sc_gather_body(att, pos, retf, idx_v, rows_v, sem):
    wid = lax.axis_index("s") * 2 + lax.axis_index("c")
    for b in range(N):
        for h in range(NH):
            pltpu.sync_copy(pos.at[b, h, pl.ds(wid * IDXR, IDXR)], idx_v)
            for j in range(IDXR):
                pltpu.async_copy(att.at[idx_v.at[j]], rows_v, sem).wait()
                dst = (b * NH + h) * L + wid * SLICE + j * 128
                pltpu.sync_copy(rows_v, retf.at[pl.ds(dst, 128)])


def _sc_gather(att, pos4):
    return pl.kernel(
        _sc_gather_body,
        out_type=jax.ShapeDtypeStruct((N * NH * L, OC), jnp.float32),
        mesh=_SC_MESH,
        scratch_types=[
            pltpu.VMEM((IDXR, 128), jnp.int32),
            pltpu.VMEM((128, OC), jnp.float32),
            pltpu.SemaphoreType.DMA,
        ],
    )(att, pos4)


CB = 512


def _comb_body(retf_ref, out_ref):
    lane = lax.broadcasted_iota(jnp.int32, (CB, OC), 1)
    msk = (lane == CH).astype(jnp.float32)
    rows = [retf_ref[0, h] for h in range(NH)]
    bs = [jnp.sum(r * msk, axis=1, keepdims=True) for r in rows]
    m = bs[0]
    for h in range(1, NH):
        m = jnp.maximum(m, bs[h])
    es = [jnp.exp(b - m) for b in bs]
    s = es[0]
    for h in range(1, NH):
        s = s + es[h]
    acc = jnp.zeros((CB, CH), jnp.float32)
    for h in range(NH):
        acc = acc + rows[h][:, :CH] * (es[h] / s)
    out_ref[0] = acc


def _combine(retf):
    return pl.pallas_call(
        _comb_body,
        grid=(N, L // CB),
        in_specs=[pl.BlockSpec((1, NH, CB, OC), lambda b, t: (b, 0, t, 0))],
        out_specs=pl.BlockSpec((1, CB, CH), lambda b, t: (b, t, 0)),
        out_shape=jax.ShapeDtypeStruct((N, L, CH), jnp.float32),
    )(retf)


def kernel(input1, input2, w_match, w_asm, w_asm_fc, fc_w1, fc_b1, fc_w2,
           fc_b2, rotations):
    n = input1.shape[0]
    hw = H * W

    x1 = _conv(input1, w_match).reshape(n, C, hw).transpose(0, 2, 1)
    x2 = _conv(input2, w_match).reshape(n, C, hw).transpose(0, 2, 1)
    x_embed = jnp.concatenate([x1, x2], axis=1)
    y1 = _conv(input1, w_asm).reshape(n, CH, hw).transpose(0, 2, 1)
    y2 = _conv(input2, w_asm).reshape(n, CH, hw).transpose(0, 2, 1)
    y_embed = jnp.concatenate([y1, y2], axis=1)
    f1 = _conv(input1, w_asm_fc).reshape(n, CH, hw).transpose(0, 2, 1)
    f2 = _conv(input2, w_asm_fc).reshape(n, CH, hw).transpose(0, 2, 1)
    fc_embed = jnp.concatenate([f1, f2], axis=1)

    hdn = jax.nn.relu(fc_embed @ fc_w1.T + fc_b1)
    fco = hdn @ fc_w2.T + fc_b2

    rotated = jnp.einsum('btf,fhi->bhti', x_embed, rotations)
    onehot = jax.nn.one_hot(
        jnp.argmax(rotated, axis=-1), HBP, dtype=jnp.float32)

    pos4 = _positions(onehot).reshape(N, NH, L // 128, 128)

    nrm = jnp.maximum(
        jnp.sqrt(jnp.sum(x_embed * x_embed, axis=-1, keepdims=True)), 5e-05)
    tok = jnp.concatenate(
        [fco, y_embed, x_embed, x_embed / nrm,
         jnp.zeros((n, L, PAYW - XN0 - C), jnp.float32)],
        axis=-1).reshape(n * L, PAYW)
    srt = _sc_scatter(tok, pos4)

    att = _chunked_attention(srt.reshape(BH, LP, PAYW))

    retf = _sc_gather(att.reshape(BH * LP, OC), pos4)
    ret = _combine(retf.reshape(n, NH, L, OC))

    out1 = ret[:, :hw, :].transpose(0, 2, 1).reshape(n, CH, H, W) + input1
    out2 = ret[:, hw:, :].transpose(0, 2, 1).reshape(n, CH, H, W) + input2
    return (out1, out2)
